# Initial kernel scaffold; baseline (speedup 1.0000x reference)
import jax
import jax.numpy as jnp
from jax.experimental import pallas as pl


def kernel(x, edge_index, W0, b0, W1, b1, W2, b2, prelu_a):
    raise NotImplementedError("write your pallas kernel here")


# speedup vs baseline: 5.5275x; 5.5275x over previous
import functools

import jax
import jax.numpy as jnp
from jax import lax
from jax.experimental import pallas as pl
from jax.experimental.pallas import tpu as pltpu
from jax.experimental.pallas import tpu_sc as plsc

N = 10000
E_RAW = 160000
E_REAL = E_RAW + N
NUM_TILES = 16
NUM_CORES = 2
K = 128
STEPS = 84
EPT = K * STEPS
E_PAD = EPT * NUM_TILES
ACC_ROWS = 10112
DUMMY = N
DEGW = 8

_MESH = plsc.VectorSubcoreMesh(core_axis_name="c", subcore_axis_name="s")


def _make_prop(C, Fc):
    CPC = C // NUM_CORES

    @functools.partial(
        pl.kernel,
        out_type=jax.ShapeDtypeStruct((C, ACC_ROWS, Fc), jnp.float32),
        mesh=_MESH,
        scratch_types=[
            pltpu.VMEM((K,), jnp.int32),
            pltpu.VMEM((K,), jnp.int32),
            pltpu.VMEM((K, Fc), jnp.float32),
            pltpu.VMEM_SHARED((ACC_ROWS, Fc), jnp.float32),
            pltpu.SemaphoreType.DMA,
        ],
    )
    def prop(table, rows, cols, zeros, out, idx_v, cidx_v, val_v, acc, sem):
        cid = lax.axis_index("c")
        sid = lax.axis_index("s")
        zr = ACC_ROWS // NUM_TILES
        for j in range(CPC):
            c = cid * CPC + j
            pltpu.sync_copy(zeros.at[pl.ds(sid * zr, zr)],
                            acc.at[pl.ds(sid * zr, zr)])
            plsc.subcore_barrier()
            ebase = c * E_PAD + sid * EPT
            cbase = sid * EPT

            def step(s, carry):
                pltpu.sync_copy(rows.at[pl.ds(ebase + s * K, K)], idx_v)
                pltpu.async_copy(table.at[idx_v], val_v, sem).wait()
                pltpu.sync_copy(cols.at[pl.ds(cbase + s * K, K)], cidx_v)
                pltpu.sync_copy(val_v, acc.at[cidx_v], add=True)
                return carry

            lax.fori_loop(0, STEPS, step, 0)
            plsc.subcore_barrier()
            pltpu.sync_copy(acc.at[pl.ds(sid * zr, zr)],
                            out.at[c, pl.ds(sid * zr, zr)])
            plsc.subcore_barrier()

    return prop


@functools.partial(
    pl.kernel,
    out_type=jax.ShapeDtypeStruct((2, ACC_ROWS, DEGW), jnp.float32),
    mesh=_MESH,
    scratch_types=[
        pltpu.VMEM((K,), jnp.int32),
        pltpu.VMEM((K, DEGW), jnp.float32),
        pltpu.VMEM_SHARED((ACC_ROWS, DEGW), jnp.float32),
        pltpu.SemaphoreType.DMA,
    ],
)
def _deg(idx2, ones_h, zeros_h, out, cidx_v, ones_v, acc, sem):
    cid = lax.axis_index("c")
    sid = lax.axis_index("s")
    zr = ACC_ROWS // NUM_TILES
    pltpu.sync_copy(ones_h, ones_v)
    pltpu.sync_copy(zeros_h.at[pl.ds(sid * zr, zr)],
                    acc.at[pl.ds(sid * zr, zr)])
    plsc.subcore_barrier()
    cbase = sid * EPT

    def step(s, carry):
        pltpu.sync_copy(idx2.at[cid, pl.ds(cbase + s * K, K)], cidx_v)
        pltpu.sync_copy(ones_v, acc.at[cidx_v], add=True)
        return carry

    lax.fori_loop(0, STEPS, step, 0)
    plsc.subcore_barrier()
    pltpu.sync_copy(acc.at[pl.ds(sid * zr, zr)],
                    out.at[cid, pl.ds(sid * zr, zr)])


@functools.partial(
    pl.kernel,
    out_type=jax.ShapeDtypeStruct((2, ACC_ROWS, 128), jnp.float32),
    mesh=_MESH,
    scratch_types=[
        pltpu.VMEM((K,), jnp.int32),
        pltpu.VMEM((K,), jnp.int32),
        pltpu.VMEM((K, 128), jnp.float32),
        pltpu.VMEM_SHARED((ACC_ROWS, 128), jnp.float32),
        pltpu.SemaphoreType.DMA,
    ],
)
def _prop_split(table, rows, cols, zeros, out, idx_v, cidx_v, val_v, acc, sem):
    cid = lax.axis_index("c")
    sid = lax.axis_index("s")
    zr = ACC_ROWS // NUM_TILES
    half = E_PAD // 2
    spt = STEPS // 2
    pltpu.sync_copy(zeros.at[pl.ds(sid * zr, zr)],
                    acc.at[pl.ds(sid * zr, zr)])
    plsc.subcore_barrier()
    ebase = cid * half + sid * (EPT // 2)

    def step(s, carry):
        pltpu.sync_copy(rows.at[pl.ds(ebase + s * K, K)], idx_v)
        pltpu.async_copy(table.at[idx_v], val_v, sem).wait()
        pltpu.sync_copy(cols.at[pl.ds(ebase + s * K, K)], cidx_v)
        pltpu.sync_copy(val_v, acc.at[cidx_v], add=True)
        return carry

    lax.fori_loop(0, spt, step, 0)
    plsc.subcore_barrier()
    pltpu.sync_copy(acc.at[pl.ds(sid * zr, zr)],
                    out.at[cid, pl.ds(sid * zr, zr)])


def _mm(x, Wt, b2d, dcol, drow, a2d, *, prologue):
    n, din = x.shape
    dout = Wt.shape[1]
    BN = 1000

    def body(x_ref, wt_ref, b_ref, dc_ref, dr_ref, a_ref, o_ref):
        xb = x_ref[...]
        if prologue:
            xb = xb * dc_ref[...] ** -1.5
            a = a_ref[0, 0]
            xb = jnp.where(xb >= 0, xb, a * xb)
        y = jnp.dot(xb, wt_ref[...], preferred_element_type=jnp.float32)
        y = (y + b_ref[...]) * lax.rsqrt(dr_ref[...])
        o_ref[...] = y

    return pl.pallas_call(
        body,
        grid=(n // BN,),
        in_specs=[
            pl.BlockSpec((BN, din), lambda i: (i, 0)),
            pl.BlockSpec((din, dout), lambda i: (0, 0)),
            pl.BlockSpec((1, dout), lambda i: (0, 0)),
            pl.BlockSpec((BN, 1), lambda i: (i, 0)),
            pl.BlockSpec((BN, 1), lambda i: (i, 0)),
            pl.BlockSpec((1, 1), lambda i: (0, 0)),
        ],
        out_specs=pl.BlockSpec((BN, dout), lambda i: (i, 0)),
        out_shape=jax.ShapeDtypeStruct((n, dout), jnp.float32),
    )(x, Wt, b2d, dcol, drow, a2d)


def _scale_sum(p, dcol):
    _, n, f = p.shape
    BN = 1000

    def body(p_ref, dc_ref, o_ref):
        o_ref[...] = (p_ref[0] + p_ref[1]) * dc_ref[...] ** -1.5

    return pl.pallas_call(
        body,
        grid=(n // BN,),
        in_specs=[
            pl.BlockSpec((2, BN, f), lambda i: (0, i, 0)),
            pl.BlockSpec((BN, 1), lambda i: (i, 0)),
        ],
        out_specs=pl.BlockSpec((BN, f), lambda i: (i, 0)),
        out_shape=jax.ShapeDtypeStruct((n, f), jnp.float32),
    )(p, dcol)


_prop4 = _make_prop(4, 128)


def _chunked(y, C, Fc):
    return y.reshape(N, C, Fc).transpose(1, 0, 2).reshape(C * N, Fc)


def _unchunk(t):
    C, _, Fc = t.shape
    return t[:, :N].transpose(1, 0, 2).reshape(N, C * Fc)


def kernel(x, edge_index, W0, b0, W1, b1, W2, b2, prelu_a):
    loop = jnp.arange(N, dtype=jnp.int32)
    row = jnp.concatenate([edge_index[0], loop])
    col = jnp.concatenate([edge_index[1], loop])
    pad = E_PAD - E_REAL
    rows_p = jnp.concatenate([row, jnp.zeros((pad,), jnp.int32)])
    cols_p = jnp.concatenate([col, jnp.full((pad,), DUMMY, jnp.int32)])
    idx2 = jnp.stack([rows_p, cols_p])

    degs = _deg(idx2,
                jnp.ones((K, DEGW), jnp.float32),
                jnp.zeros((ACC_ROWS, DEGW), jnp.float32))
    drow = degs[0, :N, 0:1]
    dcol = degs[1, :N, 0:1]
    a2d = prelu_a.reshape(1, 1)

    rows4 = (rows_p[None, :]
             + (jnp.arange(4, dtype=jnp.int32) * N)[:, None]).reshape(-1)
    z128 = jnp.zeros((ACC_ROWS, 128), jnp.float32)

    y0 = _mm(x, W0.T, b0.reshape(1, -1), dcol, drow, a2d, prologue=False)
    h0 = _unchunk(_prop4(_chunked(y0, 4, 128), rows4, cols_p, z128))
    y1 = _mm(h0, W1.T, b1.reshape(1, -1), dcol, drow, a2d, prologue=True)
    h1 = _unchunk(_prop4(_chunked(y1, 4, 128), rows4, cols_p, z128))
    y2 = _mm(h1, W2.T, b2.reshape(1, -1), dcol, drow, a2d, prologue=True)
    y2p = jnp.pad(y2, ((0, 0), (0, 64)))
    parts = _prop_split(y2p, rows_p, cols_p, z128)
    return _scale_sum(parts[:, :N, :64], dcol)


# speedup vs baseline: 7.9388x; 1.4362x over previous
import functools

import jax
import jax.numpy as jnp
from jax import lax
from jax.experimental import pallas as pl
from jax.experimental.pallas import tpu as pltpu
from jax.experimental.pallas import tpu_sc as plsc

N = 10000
E_RAW = 160000
E_REAL = E_RAW + N
NUM_TILES = 16
NUM_CORES = 2
K = 128
STEPS = 84
EPT = K * STEPS
E_PAD = EPT * NUM_TILES
ACC_ROWS = 10112
DUMMY = N
DEGW = 8

_MESH = plsc.VectorSubcoreMesh(core_axis_name="c", subcore_axis_name="s")


NBUF = 2


def _edge_pipeline(table, acc, rows_hbm, cidx, ridx, val, sems, steps, nbuf):
    def start(b, s):
        pltpu.sync_copy(rows_hbm.at[s], ridx.at[b])
        pltpu.async_copy(table.at[ridx.at[b]], val.at[b], sems[b])

    def finish(b, s):
        pltpu.make_async_copy(table.at[ridx.at[b]], val.at[b], sems[b]).wait()
        pltpu.sync_copy(val.at[b], acc.at[cidx.at[s]], add=True)

    for b in range(nbuf):
        start(b, b)

    def body(g, carry):
        base = g * nbuf
        for b in range(nbuf):
            finish(b, base + b)
            start(b, base + b + nbuf)
        return carry

    lax.fori_loop(0, steps // nbuf - 1, body, 0)
    for b in range(nbuf):
        finish(b, steps - nbuf + b)


def _make_prop(C, Fc):
    CPC = C // NUM_CORES

    @functools.partial(
        pl.kernel,
        out_type=jax.ShapeDtypeStruct((C, ACC_ROWS, Fc), jnp.float32),
        mesh=_MESH,
        scratch_types=[
            pltpu.VMEM((NBUF, K), jnp.int32),
            pltpu.VMEM((STEPS, K), jnp.int32),
            pltpu.VMEM((NBUF, K, Fc), jnp.float32),
            pltpu.VMEM_SHARED((ACC_ROWS, Fc), jnp.float32),
            pltpu.SemaphoreType.DMA,
            pltpu.SemaphoreType.DMA,
        ],
    )
    def prop(table, rows3, cols3, zeros, out,
             ridx, cidx, val, acc, s0, s1):
        cid = lax.axis_index("c")
        sid = lax.axis_index("s")
        sems = (s0, s1)
        zr = ACC_ROWS // NUM_TILES
        pltpu.sync_copy(cols3.at[sid], cidx)
        for j in range(CPC):
            c = cid * CPC + j
            rows_hbm = rows3.at[c * NUM_TILES + sid]
            pltpu.sync_copy(zeros.at[pl.ds(sid * zr, zr)],
                            acc.at[pl.ds(sid * zr, zr)])
            plsc.subcore_barrier()
            _edge_pipeline(table, acc, rows_hbm, cidx, ridx, val,
                           sems, STEPS, NBUF)
            plsc.subcore_barrier()
            pltpu.sync_copy(acc.at[pl.ds(sid * zr, zr)],
                            out.at[c, pl.ds(sid * zr, zr)])
            plsc.subcore_barrier()

    return prop


@functools.partial(
    pl.kernel,
    out_type=jax.ShapeDtypeStruct((2, ACC_ROWS, DEGW), jnp.float32),
    mesh=_MESH,
    scratch_types=[
        pltpu.VMEM((K,), jnp.int32),
        pltpu.VMEM((K, DEGW), jnp.float32),
        pltpu.VMEM_SHARED((ACC_ROWS, DEGW), jnp.float32),
        pltpu.SemaphoreType.DMA,
    ],
)
def _deg(idx2, ones_h, zeros_h, out, cidx_v, ones_v, acc, sem):
    cid = lax.axis_index("c")
    sid = lax.axis_index("s")
    zr = ACC_ROWS // NUM_TILES
    pltpu.sync_copy(ones_h, ones_v)
    pltpu.sync_copy(zeros_h.at[pl.ds(sid * zr, zr)],
                    acc.at[pl.ds(sid * zr, zr)])
    plsc.subcore_barrier()
    cbase = sid * EPT

    def step(s, carry):
        pltpu.sync_copy(idx2.at[cid, pl.ds(cbase + s * K, K)], cidx_v)
        pltpu.sync_copy(ones_v, acc.at[cidx_v], add=True)
        return carry

    lax.fori_loop(0, STEPS, step, 0)
    plsc.subcore_barrier()
    pltpu.sync_copy(acc.at[pl.ds(sid * zr, zr)],
                    out.at[cid, pl.ds(sid * zr, zr)])


SPLIT_STEPS = STEPS // 2


@functools.partial(
    pl.kernel,
    out_type=jax.ShapeDtypeStruct((2, ACC_ROWS, 128), jnp.float32),
    mesh=_MESH,
    scratch_types=[
        pltpu.VMEM((NBUF, K), jnp.int32),
        pltpu.VMEM((SPLIT_STEPS, K), jnp.int32),
        pltpu.VMEM((NBUF, K, 128), jnp.float32),
        pltpu.VMEM_SHARED((ACC_ROWS, 128), jnp.float32),
        pltpu.SemaphoreType.DMA,
        pltpu.SemaphoreType.DMA,
    ],
)
def _prop_split(table, rows3, cols3, zeros, out,
                ridx, cidx, val, acc, s0, s1):
    cid = lax.axis_index("c")
    sid = lax.axis_index("s")
    zr = ACC_ROWS // NUM_TILES
    w = cid * NUM_TILES + sid
    pltpu.sync_copy(cols3.at[w], cidx)
    pltpu.sync_copy(zeros.at[pl.ds(sid * zr, zr)],
                    acc.at[pl.ds(sid * zr, zr)])
    plsc.subcore_barrier()
    _edge_pipeline(table, acc, rows3.at[w], cidx, ridx, val,
                   (s0, s1), SPLIT_STEPS, NBUF)
    plsc.subcore_barrier()
    pltpu.sync_copy(acc.at[pl.ds(sid * zr, zr)],
                    out.at[cid, pl.ds(sid * zr, zr)])


def _mm(x, Wt, b2d, dcol, drow, a2d, *, prologue):
    n, din = x.shape
    dout = Wt.shape[1]
    BN = 1000

    def body(x_ref, wt_ref, b_ref, dc_ref, dr_ref, a_ref, o_ref):
        xb = x_ref[...]
        if prologue:
            xb = xb * dc_ref[...] ** -1.5
            a = a_ref[0, 0]
            xb = jnp.where(xb >= 0, xb, a * xb)
        y = jnp.dot(xb, wt_ref[...], preferred_element_type=jnp.float32)
        y = (y + b_ref[...]) * lax.rsqrt(dr_ref[...])
        o_ref[...] = y

    return pl.pallas_call(
        body,
        grid=(n // BN,),
        in_specs=[
            pl.BlockSpec((BN, din), lambda i: (i, 0)),
            pl.BlockSpec((din, dout), lambda i: (0, 0)),
            pl.BlockSpec((1, dout), lambda i: (0, 0)),
            pl.BlockSpec((BN, 1), lambda i: (i, 0)),
            pl.BlockSpec((BN, 1), lambda i: (i, 0)),
            pl.BlockSpec((1, 1), lambda i: (0, 0)),
        ],
        out_specs=pl.BlockSpec((BN, dout), lambda i: (i, 0)),
        out_shape=jax.ShapeDtypeStruct((n, dout), jnp.float32),
    )(x, Wt, b2d, dcol, drow, a2d)


def _scale_sum(p, dcol):
    _, n, f = p.shape
    BN = 1000

    def body(p_ref, dc_ref, o_ref):
        o_ref[...] = (p_ref[0] + p_ref[1]) * dc_ref[...] ** -1.5

    return pl.pallas_call(
        body,
        grid=(n // BN,),
        in_specs=[
            pl.BlockSpec((2, BN, f), lambda i: (0, i, 0)),
            pl.BlockSpec((BN, 1), lambda i: (i, 0)),
        ],
        out_specs=pl.BlockSpec((BN, f), lambda i: (i, 0)),
        out_shape=jax.ShapeDtypeStruct((n, f), jnp.float32),
    )(p, dcol)


_prop4 = _make_prop(4, 128)


def _chunked(y, C, Fc):
    return y.reshape(N, C, Fc).transpose(1, 0, 2).reshape(C * N, Fc)


def _unchunk(t):
    C, _, Fc = t.shape
    return t[:, :N].transpose(1, 0, 2).reshape(N, C * Fc)


def kernel(x, edge_index, W0, b0, W1, b1, W2, b2, prelu_a):
    loop = jnp.arange(N, dtype=jnp.int32)
    row = jnp.concatenate([edge_index[0], loop])
    col = jnp.concatenate([edge_index[1], loop])
    pad = E_PAD - E_REAL
    rows_p = jnp.concatenate([row, jnp.zeros((pad,), jnp.int32)])
    cols_p = jnp.concatenate([col, jnp.full((pad,), DUMMY, jnp.int32)])
    idx2 = jnp.stack([rows_p, cols_p])

    degs = _deg(idx2,
                jnp.ones((K, DEGW), jnp.float32),
                jnp.zeros((ACC_ROWS, DEGW), jnp.float32))
    drow = degs[0, :N, 0:1]
    dcol = degs[1, :N, 0:1]
    a2d = prelu_a.reshape(1, 1)

    rows4 = (rows_p[None, :]
             + (jnp.arange(4, dtype=jnp.int32) * N)[:, None]
             ).reshape(4 * NUM_TILES, STEPS, K)
    cols4 = cols_p.reshape(NUM_TILES, STEPS, K)
    rows_s = rows_p.reshape(2 * NUM_TILES, SPLIT_STEPS, K)
    cols_s = cols_p.reshape(2 * NUM_TILES, SPLIT_STEPS, K)
    z128 = jnp.zeros((ACC_ROWS, 128), jnp.float32)

    y0 = _mm(x, W0.T, b0.reshape(1, -1), dcol, drow, a2d, prologue=False)
    h0 = _unchunk(_prop4(_chunked(y0, 4, 128), rows4, cols4, z128))
    y1 = _mm(h0, W1.T, b1.reshape(1, -1), dcol, drow, a2d, prologue=True)
    h1 = _unchunk(_prop4(_chunked(y1, 4, 128), rows4, cols4, z128))
    y2 = _mm(h1, W2.T, b2.reshape(1, -1), dcol, drow, a2d, prologue=True)
    y2p = jnp.pad(y2, ((0, 0), (0, 64)))
    parts = _prop_split(y2p, rows_s, cols_s, z128)
    return _scale_sum(parts[:, :N, :64], dcol)


# speedup vs baseline: 8.3920x; 1.0571x over previous
import functools

import jax
import jax.numpy as jnp
from jax import lax
from jax.experimental import pallas as pl
from jax.experimental.pallas import tpu as pltpu
from jax.experimental.pallas import tpu_sc as plsc

N = 10000
E_RAW = 160000
E_REAL = E_RAW + N
NUM_TILES = 16
NUM_CORES = 2
K = 128
STEPS = 84
EPT = K * STEPS
E_PAD = EPT * NUM_TILES
ACC_ROWS = 10112
DUMMY = N
DEGW = 8

_MESH = plsc.VectorSubcoreMesh(core_axis_name="c", subcore_axis_name="s")


NBUF = 2


def _edge_pipeline(table, acc, rows_hbm, cidx, iring, val, sems, steps):
    nbuf = NBUF
    groups = steps // nbuf
    isems, gsems, ssems = sems

    def fetch(g, p):
        pltpu.async_copy(rows_hbm.at[pl.ds(g * nbuf, nbuf)],
                         iring.at[p], isems[p])

    def fetch_wait(g, p):
        pltpu.make_async_copy(rows_hbm.at[pl.ds(g * nbuf, nbuf)],
                              iring.at[p], isems[p]).wait()

    def gstart(p, b):
        pltpu.async_copy(table.at[iring.at[p, b]], val.at[b], gsems[b])

    def gwait(p, b):
        pltpu.make_async_copy(table.at[iring.at[p, b]], val.at[b],
                              gsems[b]).wait()

    def sstart(b, s):
        pltpu.async_copy(val.at[b], acc.at[cidx.at[s]], ssems[b], add=True)

    def swait(b, s):
        pltpu.make_async_copy(val.at[b], acc.at[cidx.at[s]], ssems[b]).wait()

    def steady(g, p):
        pn = 1 - p
        fetch_wait(g + 1, pn)
        for b in range(nbuf):
            gwait(p, b)
            sstart(b, g * nbuf + b)
        for b in range(nbuf):
            swait(b, g * nbuf + b)
            gstart(pn, b)

        cond = g + 2 < groups
        if isinstance(cond, bool):
            if cond:
                fetch(g + 2, p)
        else:
            pl.when(cond)(lambda: fetch(g + 2, p))

    fetch(0, 0)
    fetch(1, 1)
    fetch_wait(0, 0)
    for b in range(nbuf):
        gstart(0, b)

    n_steady = groups - 1
    pairs = n_steady // 2

    def body(g2, carry):
        steady(g2 * 2, 0)
        steady(g2 * 2 + 1, 1)
        return carry

    lax.fori_loop(0, pairs, body, 0)
    for g in range(2 * pairs, n_steady):
        steady(g, g % 2)
    gl = groups - 1
    for b in range(nbuf):
        gwait(gl % 2, b)
        sstart(b, gl * nbuf + b)
    for b in range(nbuf):
        swait(b, gl * nbuf + b)


def _make_prop(C, Fc):
    CPC = C // NUM_CORES

    @functools.partial(
        pl.kernel,
        out_type=jax.ShapeDtypeStruct((C, ACC_ROWS, Fc), jnp.float32),
        mesh=_MESH,
        scratch_types=[
            pltpu.VMEM((2, NBUF, K), jnp.int32),
            pltpu.VMEM((STEPS, K), jnp.int32),
            pltpu.VMEM((NBUF, K, Fc), jnp.float32),
            pltpu.VMEM_SHARED((ACC_ROWS, Fc), jnp.float32),
            pltpu.SemaphoreType.DMA,
            pltpu.SemaphoreType.DMA,
            pltpu.SemaphoreType.DMA,
            pltpu.SemaphoreType.DMA,
            pltpu.SemaphoreType.DMA,
            pltpu.SemaphoreType.DMA,
        ],
    )
    def prop(table, rows3, cols3, zeros, out,
             iring, cidx, val, acc, i0, i1, g0, g1, sc0, sc1):
        cid = lax.axis_index("c")
        sid = lax.axis_index("s")
        sems = ((i0, i1), (g0, g1), (sc0, sc1))
        zr = ACC_ROWS // NUM_TILES
        pltpu.sync_copy(cols3.at[sid], cidx)
        for j in range(CPC):
            c = cid * CPC + j
            rows_hbm = rows3.at[c * NUM_TILES + sid]
            pltpu.sync_copy(zeros.at[pl.ds(sid * zr, zr)],
                            acc.at[pl.ds(sid * zr, zr)])
            plsc.subcore_barrier()
            _edge_pipeline(table, acc, rows_hbm, cidx, iring, val,
                           sems, STEPS)
            plsc.subcore_barrier()
            pltpu.sync_copy(acc.at[pl.ds(sid * zr, zr)],
                            out.at[c, pl.ds(sid * zr, zr)])
            plsc.subcore_barrier()

    return prop


@functools.partial(
    pl.kernel,
    out_type=jax.ShapeDtypeStruct((2, ACC_ROWS, DEGW), jnp.float32),
    mesh=_MESH,
    scratch_types=[
        pltpu.VMEM((K,), jnp.int32),
        pltpu.VMEM((K, DEGW), jnp.float32),
        pltpu.VMEM_SHARED((ACC_ROWS, DEGW), jnp.float32),
        pltpu.SemaphoreType.DMA,
    ],
)
def _deg(idx2, ones_h, zeros_h, out, cidx_v, ones_v, acc, sem):
    cid = lax.axis_index("c")
    sid = lax.axis_index("s")
    zr = ACC_ROWS // NUM_TILES
    pltpu.sync_copy(ones_h, ones_v)
    pltpu.sync_copy(zeros_h.at[pl.ds(sid * zr, zr)],
                    acc.at[pl.ds(sid * zr, zr)])
    plsc.subcore_barrier()
    cbase = sid * EPT

    def step(s, carry):
        pltpu.sync_copy(idx2.at[cid, pl.ds(cbase + s * K, K)], cidx_v)
        pltpu.sync_copy(ones_v, acc.at[cidx_v], add=True)
        return carry

    lax.fori_loop(0, STEPS, step, 0)
    plsc.subcore_barrier()
    pltpu.sync_copy(acc.at[pl.ds(sid * zr, zr)],
                    out.at[cid, pl.ds(sid * zr, zr)])


SPLIT_STEPS = STEPS // 2


@functools.partial(
    pl.kernel,
    out_type=jax.ShapeDtypeStruct((2, ACC_ROWS, DEGW), jnp.float32),
    mesh=_MESH,
    scratch_types=[
        pltpu.VMEM((K,), jnp.int32),
        pltpu.VMEM((K,), jnp.int32),
        pltpu.VMEM((K, DEGW), jnp.float32),
        pltpu.VMEM_SHARED((ACC_ROWS, DEGW), jnp.float32),
        pltpu.VMEM_SHARED((ACC_ROWS, DEGW), jnp.float32),
        pltpu.SemaphoreType.DMA,
    ],
)
def _wsum(s8, rows_s, cols_s, zeros_h, out,
          ridx_v, cidx_v, val, acc, stab, sem):
    cid = lax.axis_index("c")
    sid = lax.axis_index("s")
    zr = ACC_ROWS // NUM_TILES
    pltpu.sync_copy(zeros_h.at[pl.ds(sid * zr, zr)],
                    acc.at[pl.ds(sid * zr, zr)])
    pltpu.sync_copy(s8.at[pl.ds(sid * zr, zr)],
                    stab.at[pl.ds(sid * zr, zr)])
    plsc.subcore_barrier()
    w = cid * NUM_TILES + sid

    def step(s, carry):
        pltpu.sync_copy(rows_s.at[w, s], ridx_v)
        pltpu.sync_copy(cols_s.at[w, s], cidx_v)
        pltpu.sync_copy(stab.at[ridx_v], val)
        pltpu.sync_copy(val, acc.at[cidx_v], add=True)
        return carry

    lax.fori_loop(0, SPLIT_STEPS, step, 0)
    plsc.subcore_barrier()
    pltpu.sync_copy(acc.at[pl.ds(sid * zr, zr)],
                    out.at[cid, pl.ds(sid * zr, zr)])


@functools.partial(
    pl.kernel,
    out_type=jax.ShapeDtypeStruct((2, ACC_ROWS, 128), jnp.float32),
    mesh=_MESH,
    scratch_types=[
        pltpu.VMEM((2, NBUF, K), jnp.int32),
        pltpu.VMEM((SPLIT_STEPS, K), jnp.int32),
        pltpu.VMEM((NBUF, K, 128), jnp.float32),
        pltpu.VMEM_SHARED((ACC_ROWS, 128), jnp.float32),
        pltpu.SemaphoreType.DMA,
        pltpu.SemaphoreType.DMA,
        pltpu.SemaphoreType.DMA,
        pltpu.SemaphoreType.DMA,
        pltpu.SemaphoreType.DMA,
        pltpu.SemaphoreType.DMA,
    ],
)
def _prop_split(table, rows3, cols3, zeros, out,
                iring, cidx, val, acc, i0, i1, g0, g1, sc0, sc1):
    cid = lax.axis_index("c")
    sid = lax.axis_index("s")
    zr = ACC_ROWS // NUM_TILES
    w = cid * NUM_TILES + sid
    pltpu.sync_copy(cols3.at[w], cidx)
    pltpu.sync_copy(zeros.at[pl.ds(sid * zr, zr)],
                    acc.at[pl.ds(sid * zr, zr)])
    plsc.subcore_barrier()
    _edge_pipeline(table, acc, rows3.at[w], cidx, iring, val,
                   ((i0, i1), (g0, g1), (sc0, sc1)), SPLIT_STEPS)
    plsc.subcore_barrier()
    pltpu.sync_copy(acc.at[pl.ds(sid * zr, zr)],
                    out.at[cid, pl.ds(sid * zr, zr)])


def _mm(x, Wt, b2d, dcol, drow, a2d, *, prologue):
    n, din = x.shape
    dout = Wt.shape[1]
    BN = 1000

    def body(x_ref, wt_ref, b_ref, dc_ref, dr_ref, a_ref, o_ref):
        xb = x_ref[...]
        if prologue:
            xb = xb * dc_ref[...] ** -1.5
            a = a_ref[0, 0]
            xb = jnp.where(xb >= 0, xb, a * xb)
        y = jnp.dot(xb, wt_ref[...], preferred_element_type=jnp.float32)
        y = (y + b_ref[...]) * lax.rsqrt(dr_ref[...])
        o_ref[...] = y

    return pl.pallas_call(
        body,
        grid=(n // BN,),
        in_specs=[
            pl.BlockSpec((BN, din), lambda i: (i, 0)),
            pl.BlockSpec((din, dout), lambda i: (0, 0)),
            pl.BlockSpec((1, dout), lambda i: (0, 0)),
            pl.BlockSpec((BN, 1), lambda i: (i, 0)),
            pl.BlockSpec((BN, 1), lambda i: (i, 0)),
            pl.BlockSpec((1, 1), lambda i: (0, 0)),
        ],
        out_specs=pl.BlockSpec((BN, dout), lambda i: (i, 0)),
        out_shape=jax.ShapeDtypeStruct((n, dout), jnp.float32),
    )(x, Wt, b2d, dcol, drow, a2d)


def _prep0(x, deg0):
    n, din = x.shape
    BN = 1000

    def body(x_ref, d_ref, xs_ref, s8_ref):
        s = lax.rsqrt(d_ref[...])
        xs_ref[...] = x_ref[...] * s[:, 0:1]
        s8_ref[...] = s

    return pl.pallas_call(
        body,
        grid=(n // BN,),
        in_specs=[
            pl.BlockSpec((BN, din), lambda i: (i, 0)),
            pl.BlockSpec((BN, DEGW), lambda i: (i, 0)),
        ],
        out_specs=[
            pl.BlockSpec((BN, din), lambda i: (i, 0)),
            pl.BlockSpec((BN, DEGW), lambda i: (i, 0)),
        ],
        out_shape=[
            jax.ShapeDtypeStruct((n, din), jnp.float32),
            jax.ShapeDtypeStruct((n, DEGW), jnp.float32),
        ],
    )(x, deg0)


def _mm0(xp, Wt, b2d, dcol, w2, a2d):
    n, din = xp.shape
    dout = Wt.shape[1]
    BN = 1000

    def body(x_ref, wt_ref, b_ref, dc_ref, w_ref, a_ref, o_ref):
        dc = dc_ref[...] ** -1.5
        y = jnp.dot(x_ref[...] * dc, wt_ref[...],
                    preferred_element_type=jnp.float32)
        v = (w_ref[0] + w_ref[1]) * dc
        y = y + v * b_ref[...]
        a = a_ref[0, 0]
        o_ref[...] = jnp.where(y >= 0, y, a * y)

    return pl.pallas_call(
        body,
        grid=(n // BN,),
        in_specs=[
            pl.BlockSpec((BN, din), lambda i: (i, 0)),
            pl.BlockSpec((din, dout), lambda i: (0, 0)),
            pl.BlockSpec((1, dout), lambda i: (0, 0)),
            pl.BlockSpec((BN, 1), lambda i: (i, 0)),
            pl.BlockSpec((2, BN, 1), lambda i: (0, i, 0)),
            pl.BlockSpec((1, 1), lambda i: (0, 0)),
        ],
        out_specs=pl.BlockSpec((BN, dout), lambda i: (i, 0)),
        out_shape=jax.ShapeDtypeStruct((n, dout), jnp.float32),
    )(xp, Wt, b2d, dcol, w2, a2d)


def _scale_sum(p, dcol):
    _, n, f = p.shape
    BN = 1000

    def body(p_ref, dc_ref, o_ref):
        o_ref[...] = (p_ref[0] + p_ref[1]) * dc_ref[...] ** -1.5

    return pl.pallas_call(
        body,
        grid=(n // BN,),
        in_specs=[
            pl.BlockSpec((2, BN, f), lambda i: (0, i, 0)),
            pl.BlockSpec((BN, 1), lambda i: (i, 0)),
        ],
        out_specs=pl.BlockSpec((BN, f), lambda i: (i, 0)),
        out_shape=jax.ShapeDtypeStruct((n, f), jnp.float32),
    )(p, dcol)


_prop4 = _make_prop(4, 128)
_prop2 = _make_prop(2, 128)


def _chunked(y, C, Fc):
    return y.reshape(N, C, Fc).transpose(1, 0, 2).reshape(C * N, Fc)


def _unchunk(t):
    C, _, Fc = t.shape
    return t[:, :N].transpose(1, 0, 2).reshape(N, C * Fc)


def kernel(x, edge_index, W0, b0, W1, b1, W2, b2, prelu_a):
    loop = jnp.arange(N, dtype=jnp.int32)
    row = jnp.concatenate([edge_index[0], loop])
    col = jnp.concatenate([edge_index[1], loop])
    pad = E_PAD - E_REAL
    rows_p = jnp.concatenate([row, jnp.zeros((pad,), jnp.int32)])
    cols_p = jnp.concatenate([col, jnp.full((pad,), DUMMY, jnp.int32)])
    idx2 = jnp.stack([rows_p, cols_p])

    degs = _deg(idx2,
                jnp.ones((K, DEGW), jnp.float32),
                jnp.zeros((ACC_ROWS, DEGW), jnp.float32))
    drow = degs[0, :N, 0:1]
    dcol = degs[1, :N, 0:1]
    a2d = prelu_a.reshape(1, 1)

    rows4 = (rows_p[None, :]
             + (jnp.arange(4, dtype=jnp.int32) * N)[:, None]
             ).reshape(4 * NUM_TILES, STEPS, K)
    rows2 = (rows_p[None, :]
             + (jnp.arange(2, dtype=jnp.int32) * N)[:, None]
             ).reshape(2 * NUM_TILES, STEPS, K)
    cols4 = cols_p.reshape(NUM_TILES, STEPS, K)
    rows_s = rows_p.reshape(2 * NUM_TILES, SPLIT_STEPS, K)
    cols_s = cols_p.reshape(2 * NUM_TILES, SPLIT_STEPS, K)
    z128 = jnp.zeros((ACC_ROWS, 128), jnp.float32)
    z8 = jnp.zeros((ACC_ROWS, DEGW), jnp.float32)

    xs, s8 = _prep0(x, degs[0, :N, :])
    s8p = jnp.pad(s8, ((0, ACC_ROWS - N), (0, 0)))
    wparts = _wsum(s8p, rows_s, cols_s, z8)
    xp = _unchunk(_prop2(_chunked(xs, 2, 128), rows2, cols4, z128))
    z0 = _mm0(xp, W0.T, b0.reshape(1, -1), dcol, wparts[:, :N, 0:1], a2d)
    y1 = _mm(z0, W1.T, b1.reshape(1, -1), dcol, drow, a2d, prologue=False)
    h1 = _unchunk(_prop4(_chunked(y1, 4, 128), rows4, cols4, z128))
    y2 = _mm(h1, W2.T, b2.reshape(1, -1), dcol, drow, a2d, prologue=True)
    y2p = jnp.pad(y2, ((0, 0), (0, 64)))
    parts = _prop_split(y2p, rows_s, cols_s, z128)
    return _scale_sum(parts[:, :N, :64], dcol)


# speedup vs baseline: 8.7124x; 1.0382x over previous
import functools

import jax
import jax.numpy as jnp
from jax import lax
from jax.experimental import pallas as pl
from jax.experimental.pallas import tpu as pltpu
from jax.experimental.pallas import tpu_sc as plsc

N = 10000
E_RAW = 160000
E_REAL = E_RAW + N
NUM_TILES = 16
NUM_CORES = 2
K = 128
STEPS = 84
EPT = K * STEPS
E_PAD = EPT * NUM_TILES
ACC_ROWS = 10112
DUMMY = N
DEGW = 8

_MESH = plsc.VectorSubcoreMesh(core_axis_name="c", subcore_axis_name="s")


NBUF = 2


def _edge_pipeline(table, acc, rows_hbm, cidx, iring, val, sems, steps):
    nbuf = NBUF
    groups = steps // nbuf
    isems, gsems, ssems = sems

    def fetch(g, p):
        pltpu.async_copy(rows_hbm.at[pl.ds(g * nbuf, nbuf)],
                         iring.at[p], isems[p])

    def fetch_wait(g, p):
        pltpu.make_async_copy(rows_hbm.at[pl.ds(g * nbuf, nbuf)],
                              iring.at[p], isems[p]).wait()

    def gstart(p, b):
        pltpu.async_copy(table.at[iring.at[p, b]], val.at[b], gsems[b])

    def gwait(p, b):
        pltpu.make_async_copy(table.at[iring.at[p, b]], val.at[b],
                              gsems[b]).wait()

    def sstart(b, s):
        pltpu.async_copy(val.at[b], acc.at[cidx.at[s]], ssems[b], add=True)

    def swait(b, s):
        pltpu.make_async_copy(val.at[b], acc.at[cidx.at[s]], ssems[b]).wait()

    def steady(g, p):
        pn = 1 - p
        fetch_wait(g + 1, pn)
        for b in range(nbuf):
            gwait(p, b)
            sstart(b, g * nbuf + b)
        for b in range(nbuf):
            swait(b, g * nbuf + b)
            gstart(pn, b)

        cond = g + 2 < groups
        if isinstance(cond, bool):
            if cond:
                fetch(g + 2, p)
        else:
            pl.when(cond)(lambda: fetch(g + 2, p))

    fetch(0, 0)
    fetch(1, 1)
    fetch_wait(0, 0)
    for b in range(nbuf):
        gstart(0, b)

    n_steady = groups - 1
    pairs = n_steady // 2

    def body(g2, carry):
        steady(g2 * 2, 0)
        steady(g2 * 2 + 1, 1)
        return carry

    lax.fori_loop(0, pairs, body, 0)
    for g in range(2 * pairs, n_steady):
        steady(g, g % 2)
    gl = groups - 1
    for b in range(nbuf):
        gwait(gl % 2, b)
        sstart(b, gl * nbuf + b)
    for b in range(nbuf):
        swait(b, gl * nbuf + b)


def _make_prop(C, Fc):
    CPC = C // NUM_CORES

    @functools.partial(
        pl.kernel,
        out_type=jax.ShapeDtypeStruct((C, ACC_ROWS, Fc), jnp.float32),
        mesh=_MESH,
        scratch_types=[
            pltpu.VMEM((2, NBUF, K), jnp.int32),
            pltpu.VMEM((STEPS, K), jnp.int32),
            pltpu.VMEM((NBUF, K, Fc), jnp.float32),
            pltpu.VMEM_SHARED((ACC_ROWS, Fc), jnp.float32),
            pltpu.SemaphoreType.DMA,
            pltpu.SemaphoreType.DMA,
            pltpu.SemaphoreType.DMA,
            pltpu.SemaphoreType.DMA,
            pltpu.SemaphoreType.DMA,
            pltpu.SemaphoreType.DMA,
        ],
    )
    def prop(table, rows3, cols3, zeros, out,
             iring, cidx, val, acc, i0, i1, g0, g1, sc0, sc1):
        cid = lax.axis_index("c")
        sid = lax.axis_index("s")
        sems = ((i0, i1), (g0, g1), (sc0, sc1))
        zr = ACC_ROWS // NUM_TILES
        pltpu.sync_copy(cols3.at[sid], cidx)
        for j in range(CPC):
            c = cid * CPC + j
            rows_hbm = rows3.at[c * NUM_TILES + sid]
            pltpu.sync_copy(zeros.at[pl.ds(sid * zr, zr)],
                            acc.at[pl.ds(sid * zr, zr)])
            plsc.subcore_barrier()
            _edge_pipeline(table, acc, rows_hbm, cidx, iring, val,
                           sems, STEPS)
            plsc.subcore_barrier()
            pltpu.sync_copy(acc.at[pl.ds(sid * zr, zr)],
                            out.at[c, pl.ds(sid * zr, zr)])
            plsc.subcore_barrier()

    return prop


@functools.partial(
    pl.kernel,
    out_type=jax.ShapeDtypeStruct((2, ACC_ROWS, DEGW), jnp.float32),
    mesh=_MESH,
    scratch_types=[
        pltpu.VMEM((K,), jnp.int32),
        pltpu.VMEM((K, DEGW), jnp.float32),
        pltpu.VMEM_SHARED((ACC_ROWS, DEGW), jnp.float32),
        pltpu.SemaphoreType.DMA,
    ],
)
def _deg(idx2, ones_h, zeros_h, out, cidx_v, ones_v, acc, sem):
    cid = lax.axis_index("c")
    sid = lax.axis_index("s")
    zr = ACC_ROWS // NUM_TILES
    pltpu.sync_copy(ones_h, ones_v)
    pltpu.sync_copy(zeros_h.at[pl.ds(sid * zr, zr)],
                    acc.at[pl.ds(sid * zr, zr)])
    plsc.subcore_barrier()
    cbase = sid * EPT

    def step(s, carry):
        pltpu.sync_copy(idx2.at[cid, pl.ds(cbase + s * K, K)], cidx_v)
        pltpu.sync_copy(ones_v, acc.at[cidx_v], add=True)
        return carry

    lax.fori_loop(0, STEPS, step, 0)
    plsc.subcore_barrier()
    pltpu.sync_copy(acc.at[pl.ds(sid * zr, zr)],
                    out.at[cid, pl.ds(sid * zr, zr)])


SPLIT_STEPS = STEPS // 2


@functools.partial(
    pl.kernel,
    out_type=jax.ShapeDtypeStruct((2, ACC_ROWS, DEGW), jnp.float32),
    mesh=_MESH,
    scratch_types=[
        pltpu.VMEM((SPLIT_STEPS, K), jnp.int32),
        pltpu.VMEM((SPLIT_STEPS, K), jnp.int32),
        pltpu.VMEM((2, K, DEGW), jnp.float32),
        pltpu.VMEM_SHARED((ACC_ROWS, DEGW), jnp.float32),
        pltpu.VMEM_SHARED((ACC_ROWS, DEGW), jnp.float32),
        pltpu.SemaphoreType.DMA,
        pltpu.SemaphoreType.DMA,
    ],
)
def _wsum(s8, rows_s, cols_s, zeros_h, out,
          ridx, cidx, val, acc, stab, g0, g1):
    cid = lax.axis_index("c")
    sid = lax.axis_index("s")
    zr = ACC_ROWS // NUM_TILES
    pltpu.sync_copy(zeros_h.at[pl.ds(sid * zr, zr)],
                    acc.at[pl.ds(sid * zr, zr)])
    pltpu.sync_copy(s8.at[pl.ds(sid * zr, zr)],
                    stab.at[pl.ds(sid * zr, zr)])
    w = cid * NUM_TILES + sid
    pltpu.sync_copy(rows_s.at[w], ridx)
    pltpu.sync_copy(cols_s.at[w], cidx)
    plsc.subcore_barrier()
    gsems = (g0, g1)

    def gstart(s, b):
        pltpu.async_copy(stab.at[ridx.at[s]], val.at[b], gsems[b])

    def gwait(s, b):
        pltpu.make_async_copy(stab.at[ridx.at[s]], val.at[b],
                              gsems[b]).wait()

    def scat(s, b):
        pltpu.sync_copy(val.at[b], acc.at[cidx.at[s]], add=True)

    gstart(0, 0)
    gstart(1, 1)
    pairs = SPLIT_STEPS // 2

    def body(g, carry):
        s0 = 2 * g
        gwait(s0, 0)
        scat(s0, 0)
        gstart(s0 + 2, 0)
        gwait(s0 + 1, 1)
        scat(s0 + 1, 1)
        gstart(s0 + 3, 1)
        return carry

    lax.fori_loop(0, pairs - 1, body, 0)
    sl = SPLIT_STEPS - 2
    gwait(sl, 0)
    scat(sl, 0)
    gwait(sl + 1, 1)
    scat(sl + 1, 1)
    plsc.subcore_barrier()
    pltpu.sync_copy(acc.at[pl.ds(sid * zr, zr)],
                    out.at[cid, pl.ds(sid * zr, zr)])


@functools.partial(
    pl.kernel,
    out_type=jax.ShapeDtypeStruct((2, ACC_ROWS, 128), jnp.float32),
    mesh=_MESH,
    scratch_types=[
        pltpu.VMEM((2, NBUF, K), jnp.int32),
        pltpu.VMEM((SPLIT_STEPS, K), jnp.int32),
        pltpu.VMEM((NBUF, K, 128), jnp.float32),
        pltpu.VMEM_SHARED((ACC_ROWS, 128), jnp.float32),
        pltpu.SemaphoreType.DMA,
        pltpu.SemaphoreType.DMA,
        pltpu.SemaphoreType.DMA,
        pltpu.SemaphoreType.DMA,
        pltpu.SemaphoreType.DMA,
        pltpu.SemaphoreType.DMA,
    ],
)
def _prop_split(table, rows3, cols3, zeros, out,
                iring, cidx, val, acc, i0, i1, g0, g1, sc0, sc1):
    cid = lax.axis_index("c")
    sid = lax.axis_index("s")
    zr = ACC_ROWS // NUM_TILES
    w = cid * NUM_TILES + sid
    pltpu.sync_copy(cols3.at[w], cidx)
    pltpu.sync_copy(zeros.at[pl.ds(sid * zr, zr)],
                    acc.at[pl.ds(sid * zr, zr)])
    plsc.subcore_barrier()
    _edge_pipeline(table, acc, rows3.at[w], cidx, iring, val,
                   ((i0, i1), (g0, g1), (sc0, sc1)), SPLIT_STEPS)
    plsc.subcore_barrier()
    pltpu.sync_copy(acc.at[pl.ds(sid * zr, zr)],
                    out.at[cid, pl.ds(sid * zr, zr)])


def _mm(x, Wt, b2d, dcol, drow, a2d, *, prologue):
    n, din = x.shape
    dout = Wt.shape[1]
    BN = 1000

    def body(x_ref, wt_ref, b_ref, dc_ref, dr_ref, a_ref, o_ref):
        xb = x_ref[...]
        if prologue:
            xb = xb * dc_ref[...] ** -1.5
            a = a_ref[0, 0]
            xb = jnp.where(xb >= 0, xb, a * xb)
        y = jnp.dot(xb, wt_ref[...], preferred_element_type=jnp.float32)
        y = (y + b_ref[...]) * lax.rsqrt(dr_ref[...])
        o_ref[...] = y

    return pl.pallas_call(
        body,
        grid=(n // BN,),
        in_specs=[
            pl.BlockSpec((BN, din), lambda i: (i, 0)),
            pl.BlockSpec((din, dout), lambda i: (0, 0)),
            pl.BlockSpec((1, dout), lambda i: (0, 0)),
            pl.BlockSpec((BN, 1), lambda i: (i, 0)),
            pl.BlockSpec((BN, 1), lambda i: (i, 0)),
            pl.BlockSpec((1, 1), lambda i: (0, 0)),
        ],
        out_specs=pl.BlockSpec((BN, dout), lambda i: (i, 0)),
        out_shape=jax.ShapeDtypeStruct((n, dout), jnp.float32),
    )(x, Wt, b2d, dcol, drow, a2d)


def _prep0(x, deg0):
    n, din = x.shape
    BN = 1000

    def body(x_ref, d_ref, xs_ref, s8_ref):
        s = lax.rsqrt(d_ref[...])
        xs_ref[...] = x_ref[...] * s[:, 0:1]
        s8_ref[...] = s

    return pl.pallas_call(
        body,
        grid=(n // BN,),
        in_specs=[
            pl.BlockSpec((BN, din), lambda i: (i, 0)),
            pl.BlockSpec((BN, DEGW), lambda i: (i, 0)),
        ],
        out_specs=[
            pl.BlockSpec((BN, din), lambda i: (i, 0)),
            pl.BlockSpec((BN, DEGW), lambda i: (i, 0)),
        ],
        out_shape=[
            jax.ShapeDtypeStruct((n, din), jnp.float32),
            jax.ShapeDtypeStruct((n, DEGW), jnp.float32),
        ],
    )(x, deg0)


def _mm0(xp, Wt, b2d, dcol, w2, a2d):
    n, din = xp.shape
    dout = Wt.shape[1]
    BN = 1000

    def body(x_ref, wt_ref, b_ref, dc_ref, w_ref, a_ref, o_ref):
        dc = dc_ref[...] ** -1.5
        y = jnp.dot(x_ref[...] * dc, wt_ref[...],
                    preferred_element_type=jnp.float32)
        v = (w_ref[0] + w_ref[1]) * dc
        y = y + v * b_ref[...]
        a = a_ref[0, 0]
        o_ref[...] = jnp.where(y >= 0, y, a * y)

    return pl.pallas_call(
        body,
        grid=(n // BN,),
        in_specs=[
            pl.BlockSpec((BN, din), lambda i: (i, 0)),
            pl.BlockSpec((din, dout), lambda i: (0, 0)),
            pl.BlockSpec((1, dout), lambda i: (0, 0)),
            pl.BlockSpec((BN, 1), lambda i: (i, 0)),
            pl.BlockSpec((2, BN, 1), lambda i: (0, i, 0)),
            pl.BlockSpec((1, 1), lambda i: (0, 0)),
        ],
        out_specs=pl.BlockSpec((BN, dout), lambda i: (i, 0)),
        out_shape=jax.ShapeDtypeStruct((n, dout), jnp.float32),
    )(xp, Wt, b2d, dcol, w2, a2d)


def _scale_sum(p, dcol):
    _, n, f = p.shape
    BN = 1000

    def body(p_ref, dc_ref, o_ref):
        o_ref[...] = (p_ref[0] + p_ref[1]) * dc_ref[...] ** -1.5

    return pl.pallas_call(
        body,
        grid=(n // BN,),
        in_specs=[
            pl.BlockSpec((2, BN, f), lambda i: (0, i, 0)),
            pl.BlockSpec((BN, 1), lambda i: (i, 0)),
        ],
        out_specs=pl.BlockSpec((BN, f), lambda i: (i, 0)),
        out_shape=jax.ShapeDtypeStruct((n, f), jnp.float32),
    )(p, dcol)


_prop4 = _make_prop(4, 128)
_prop2 = _make_prop(2, 128)


def _chunked(y, C, Fc):
    return y.reshape(N, C, Fc).transpose(1, 0, 2).reshape(C * N, Fc)


def _unchunk(t):
    C, _, Fc = t.shape
    return t[:, :N].transpose(1, 0, 2).reshape(N, C * Fc)


def kernel(x, edge_index, W0, b0, W1, b1, W2, b2, prelu_a):
    loop = jnp.arange(N, dtype=jnp.int32)
    row = jnp.concatenate([edge_index[0], loop])
    col = jnp.concatenate([edge_index[1], loop])
    pad = E_PAD - E_REAL
    rows_p = jnp.concatenate([row, jnp.zeros((pad,), jnp.int32)])
    cols_p = jnp.concatenate([col, jnp.full((pad,), DUMMY, jnp.int32)])
    idx2 = jnp.stack([rows_p, cols_p])

    degs = _deg(idx2,
                jnp.ones((K, DEGW), jnp.float32),
                jnp.zeros((ACC_ROWS, DEGW), jnp.float32))
    drow = degs[0, :N, 0:1]
    dcol = degs[1, :N, 0:1]
    a2d = prelu_a.reshape(1, 1)

    rows4 = (rows_p[None, :]
             + (jnp.arange(4, dtype=jnp.int32) * N)[:, None]
             ).reshape(4 * NUM_TILES, STEPS, K)
    rows2 = (rows_p[None, :]
             + (jnp.arange(2, dtype=jnp.int32) * N)[:, None]
             ).reshape(2 * NUM_TILES, STEPS, K)
    cols4 = cols_p.reshape(NUM_TILES, STEPS, K)
    rows_s = rows_p.reshape(2 * NUM_TILES, SPLIT_STEPS, K)
    cols_s = cols_p.reshape(2 * NUM_TILES, SPLIT_STEPS, K)
    z128 = jnp.zeros((ACC_ROWS, 128), jnp.float32)
    z8 = jnp.zeros((ACC_ROWS, DEGW), jnp.float32)

    xs, s8 = _prep0(x, degs[0, :N, :])
    s8p = jnp.pad(s8, ((0, ACC_ROWS - N), (0, 0)))
    wparts = _wsum(s8p, rows_s, cols_s, z8)
    xp = _unchunk(_prop2(_chunked(xs, 2, 128), rows2, cols4, z128))
    z0 = _mm0(xp, W0.T, b0.reshape(1, -1), dcol, wparts[:, :N, 0:1], a2d)
    y1 = _mm(z0, W1.T, b1.reshape(1, -1), dcol, drow, a2d, prologue=False)
    h1 = _unchunk(_prop4(_chunked(y1, 4, 128), rows4, cols4, z128))
    y2 = _mm(h1, W2.T, b2.reshape(1, -1), dcol, drow, a2d, prologue=True)
    y2p = jnp.pad(y2, ((0, 0), (0, 64)))
    parts = _prop_split(y2p, rows_s, cols_s, z128)
    return _scale_sum(parts[:, :N, :64], dcol)
